# baseline (speedup 1.0000x reference)
import jax
import jax.numpy as jnp
from jax.experimental import pallas as pl
from jax.experimental.pallas import tpu as pltpu

_CHUNK_ROWS = (16,) * 8
_OFFSETS = tuple(sum(_CHUNK_ROWS[:i]) for i in range(len(_CHUNK_ROWS)))
_MAX_ROWS = max(_CHUNK_ROWS)
_SLOTS = 4


def _copy_body(x_hbm, o_hbm, buf, rsem, wsem):
    chunks = len(_CHUNK_ROWS)

    def read(i):
        s = i % _SLOTS
        return pltpu.make_async_copy(
            x_hbm.at[pl.ds(_OFFSETS[i], _CHUNK_ROWS[i]), :],
            buf.at[s, pl.ds(0, _CHUNK_ROWS[i])], rsem.at[s])

    def write(i):
        s = i % _SLOTS
        return pltpu.make_async_copy(
            buf.at[s, pl.ds(0, _CHUNK_ROWS[i])],
            o_hbm.at[pl.ds(_OFFSETS[i], _CHUNK_ROWS[i]), :], wsem.at[s])

    for i in range(min(_SLOTS, chunks)):
        read(i).start()
    for i in range(chunks):
        read(i).wait()
        write(i).start()
        if i + _SLOTS < chunks:
            write(i).wait()
            read(i + _SLOTS).start()
    for i in range(max(chunks - _SLOTS, 0), chunks):
        write(i).wait()


def kernel(x, relative_position_bias_table):
    del relative_position_bias_table
    b, s, d = x.shape
    n = b * s * d
    rows = 128
    x2 = x.reshape(rows, n // rows)
    out = pl.pallas_call(
        _copy_body,
        in_specs=[pl.BlockSpec(memory_space=pl.ANY)],
        out_specs=pl.BlockSpec(memory_space=pl.ANY),
        out_shape=jax.ShapeDtypeStruct(x2.shape, x.dtype),
        scratch_shapes=[
            pltpu.VMEM((_SLOTS, _MAX_ROWS, x2.shape[1]), x.dtype),
            pltpu.SemaphoreType.DMA((_SLOTS,)),
            pltpu.SemaphoreType.DMA((_SLOTS,)),
        ],
    )(x2)
    return out.reshape(b, s, d)


# speedup vs baseline: 3.5253x; 3.5253x over previous
import jax
import jax.numpy as jnp
from jax.experimental import pallas as pl
from jax.experimental.pallas import tpu as pltpu

_CHUNK_ROWS = (1024,) + (2048,) * 7 + (1024,)
_OFFSETS = tuple(sum(_CHUNK_ROWS[:i]) for i in range(len(_CHUNK_ROWS)))
_MAX_ROWS = max(_CHUNK_ROWS)
_SLOTS = 4


def _copy_body(x_hbm, o_hbm, buf, rsem, wsem):
    chunks = len(_CHUNK_ROWS)

    def read(i):
        s = i % _SLOTS
        return pltpu.make_async_copy(
            x_hbm.at[pl.ds(_OFFSETS[i], _CHUNK_ROWS[i]), :],
            buf.at[s, pl.ds(0, _CHUNK_ROWS[i])], rsem.at[s])

    def write(i):
        s = i % _SLOTS
        return pltpu.make_async_copy(
            buf.at[s, pl.ds(0, _CHUNK_ROWS[i])],
            o_hbm.at[pl.ds(_OFFSETS[i], _CHUNK_ROWS[i]), :], wsem.at[s])

    for i in range(min(_SLOTS, chunks)):
        read(i).start()
    for i in range(chunks):
        read(i).wait()
        write(i).start()
        if i + _SLOTS < chunks:
            write(i).wait()
            read(i + _SLOTS).start()
    for i in range(max(chunks - _SLOTS, 0), chunks):
        write(i).wait()


def kernel(x, relative_position_bias_table):
    del relative_position_bias_table
    b, s, d = x.shape
    n = b * s * d
    rows = 16384
    x2 = x.reshape(rows, n // rows)
    out = pl.pallas_call(
        _copy_body,
        in_specs=[pl.BlockSpec(memory_space=pl.ANY)],
        out_specs=pl.BlockSpec(memory_space=pl.ANY),
        out_shape=jax.ShapeDtypeStruct(x2.shape, x.dtype),
        scratch_shapes=[
            pltpu.VMEM((_SLOTS, _MAX_ROWS, x2.shape[1]), x.dtype),
            pltpu.SemaphoreType.DMA((_SLOTS,)),
            pltpu.SemaphoreType.DMA((_SLOTS,)),
        ],
    )(x2)
    return out.reshape(b, s, d)
